# baseline (speedup 1.0000x reference)
import jax
import jax.numpy as jnp
from jax.experimental import pallas as pl
from jax.experimental.pallas import tpu as pltpu


def _pool_ln_kernel(x_ref, w_ref, b_ref, q_ref):
    xb = x_ref[...]
    avg = jnp.mean(xb, axis=-1)
    mu = jnp.mean(avg, axis=-1, keepdims=True)
    var = jnp.mean((avg - mu) ** 2, axis=-1, keepdims=True)
    q_ref[...] = (avg - mu) / jnp.sqrt(var + 1e-5) * w_ref[...] + b_ref[...]


def _attn_argmax_kernel(q_ref, idx_ref, *, n, scale):
    q = q_ref[0]
    a = jnp.dot(q, q.T) * scale
    rows = jax.lax.broadcasted_iota(jnp.int32, (n, n), 0)
    cols = jax.lax.broadcasted_iota(jnp.int32, (n, n), 1)
    a = jnp.where(rows == cols, a - 100.0, a)
    idx_ref[0, 0] = jnp.argmax(a, axis=-1).astype(jnp.int32)


def _gather_kernel(idx_ref, src_ref, out_ref):
    del idx_ref
    out_ref[...] = src_ref[...]


def kernel(x, ln_w, ln_b):
    b, n, c, h, w = x.shape
    hw = h * w
    scale = c ** (-0.5)
    x_flat = x.reshape(b * n, c, hw)

    R = 64
    q = pl.pallas_call(
        _pool_ln_kernel,
        grid=(b * n // R,),
        in_specs=[
            pl.BlockSpec((R, c, hw), lambda i: (i, 0, 0)),
            pl.BlockSpec((1, c), lambda i: (0, 0)),
            pl.BlockSpec((1, c), lambda i: (0, 0)),
        ],
        out_specs=pl.BlockSpec((R, c), lambda i: (i, 0)),
        out_shape=jax.ShapeDtypeStruct((b * n, c), x.dtype),
    )(x_flat, ln_w.reshape(1, c), ln_b.reshape(1, c))
    q = q.reshape(b, n, c)

    import functools
    idx = pl.pallas_call(
        functools.partial(_attn_argmax_kernel, n=n, scale=scale),
        grid=(b,),
        in_specs=[pl.BlockSpec((1, n, c), lambda i: (i, 0, 0))],
        out_specs=pl.BlockSpec((1, 1, n), lambda i: (i, 0, 0)),
        out_shape=jax.ShapeDtypeStruct((b, 1, n), jnp.int32),
    )(q)
    idx_flat = idx.reshape(b * n)

    y = pl.pallas_call(
        _gather_kernel,
        grid_spec=pltpu.PrefetchScalarGridSpec(
            num_scalar_prefetch=1,
            grid=(b * n,),
            in_specs=[pl.BlockSpec((1, c, hw), lambda i, idx_ref: (idx_ref[i], 0, 0))],
            out_specs=pl.BlockSpec((1, c, hw), lambda i, idx_ref: (i, 0, 0)),
        ),
        out_shape=jax.ShapeDtypeStruct((b * n, c, hw), x.dtype),
    )(idx_flat, x_flat)
    return y.reshape(b, n, c, h, w)
